# baseline (device time: 218282 ns/iter reference)
import jax
import jax.numpy as jnp
from jax import lax
from jax.experimental import pallas as pl
from jax.experimental.pallas import tpu as pltpu

N_DEV = 4
SQ = 1024
SKV = 1024
H_TOT = 32
H_PER = 8
DH = 128
D_MODEL = 1024
CHUNK = H_PER * DH
SCALE = 0.08838834764831843
QBLK = 64


def _body(x_ref, wq_ref, k_ref, v_ref, wo_ref, out_ref,
          wq_ch, wo_ch, ctx_buf, bias_ref, k_stage, v_stage,
          sq_send, sq_recv, so_send, so_recv, kv_sems):
    my = lax.axis_index("i")
    right = lax.rem(my + 1, N_DEV)
    left = lax.rem(my + N_DEV - 1, N_DEV)

    def kv_fetch(c, slot):
        ck = pltpu.make_async_copy(
            k_ref.at[:, pl.ds(c * CHUNK, CHUNK)], k_stage.at[slot],
            kv_sems.at[slot, 0])
        cv = pltpu.make_async_copy(
            v_ref.at[:, pl.ds(c * CHUNK, CHUNK)], v_stage.at[slot],
            kv_sems.at[slot, 1])
        ck.start()
        cv.start()
        return ck, cv

    def kv_wait(c, slot):
        ck = pltpu.make_async_copy(
            k_ref.at[:, pl.ds(c * CHUNK, CHUNK)], k_stage.at[slot],
            kv_sems.at[slot, 0])
        cv = pltpu.make_async_copy(
            v_ref.at[:, pl.ds(c * CHUNK, CHUNK)], v_stage.at[slot],
            kv_sems.at[slot, 1])
        ck.wait()
        cv.wait()

    kv_fetch(my, 0)

    barrier = pltpu.get_barrier_semaphore()
    for nbr in (left, right):
        pl.semaphore_signal(barrier, inc=1, device_id=(nbr,),
                            device_id_type=pl.DeviceIdType.MESH)
    pl.semaphore_wait(barrier, 2)

    wq_ch[my] = wq_ref[...].astype(jnp.bfloat16)
    wo_ch[my] = wo_ref[...].astype(jnp.bfloat16)

    q_idx = my * SQ + lax.broadcasted_iota(jnp.int32, (SQ, SKV), 0)
    k_idx = lax.broadcasted_iota(jnp.int32, (SQ, SKV), 1)
    qb = q_idx // QBLK
    kb = k_idx // QBLK
    mask = (qb == kb) | (kb == 0) | (lax.rem(qb + kb, 3) == 0)
    bias_ref[...] = jnp.where(mask, 0.0, -1e9).astype(jnp.float32)

    out_ref[...] = jnp.zeros((SQ, D_MODEL), jnp.float32)
    xb = x_ref[...].astype(jnp.bfloat16)

    def compute_chunk(c, slot):
        def head_body(hl, _):
            wq = wq_ch[c, :, pl.ds(hl * DH, DH)]
            q = lax.dot_general(xb, wq, (((1,), (0,)), ((), ())),
                                preferred_element_type=jnp.float32)
            q = (q * SCALE).astype(jnp.bfloat16)
            k = k_stage[slot, :, pl.ds(hl * DH, DH)].astype(jnp.bfloat16)
            s = lax.dot_general(q, k, (((1,), (1,)), ((), ())),
                                preferred_element_type=jnp.float32)
            w = jnp.exp(s + bias_ref[...])
            r = 1.0 / jnp.sum(w, axis=-1, keepdims=True)
            v = v_stage[slot, :, pl.ds(hl * DH, DH)].astype(jnp.bfloat16)
            ctx = lax.dot_general(w.astype(jnp.bfloat16), v,
                                  (((1,), (0,)), ((), ())),
                                  preferred_element_type=jnp.float32)
            ctx_buf[:, pl.ds(hl * DH, DH)] = (ctx * r).astype(jnp.bfloat16)
            return 0

        lax.fori_loop(0, H_PER, head_body, 0)
        out_ref[...] += lax.dot_general(ctx_buf[...], wo_ch[c],
                                        (((1,), (0,)), ((), ())),
                                        preferred_element_type=jnp.float32)

    for h in range(N_DEV - 1):
        src = lax.rem(my - h + N_DEV, N_DEV)
        rq = pltpu.make_async_remote_copy(
            src_ref=wq_ch.at[src], dst_ref=wq_ch.at[src],
            send_sem=sq_send.at[h], recv_sem=sq_recv.at[h],
            device_id=(right,), device_id_type=pl.DeviceIdType.MESH)
        ro = pltpu.make_async_remote_copy(
            src_ref=wo_ch.at[src], dst_ref=wo_ch.at[src],
            send_sem=so_send.at[h], recv_sem=so_recv.at[h],
            device_id=(right,), device_id_type=pl.DeviceIdType.MESH)
        rq.start()
        ro.start()
        kv_wait(src, h % 2)
        kv_fetch(lax.rem(src + N_DEV - 1, N_DEV), (h + 1) % 2)
        compute_chunk(src, h % 2)
        rq.wait()
        ro.wait()

    last = lax.rem(my + 1, N_DEV)
    kv_wait(last, 1)
    compute_chunk(last, 1)


def kernel(x, Wq, K_ext, V_ext, Wo):
    kf = K_ext[0].reshape(SKV, H_TOT * DH)
    vf = V_ext[0].reshape(SKV, H_TOT * DH)

    vmem = pl.BlockSpec(memory_space=pltpu.MemorySpace.VMEM)
    hbm = pl.BlockSpec(memory_space=pltpu.MemorySpace.HBM)
    out = pl.pallas_call(
        _body,
        out_shape=jax.ShapeDtypeStruct((SQ, D_MODEL), jnp.float32),
        in_specs=[vmem, vmem, hbm, hbm, vmem],
        out_specs=vmem,
        scratch_shapes=[
            pltpu.VMEM((N_DEV, D_MODEL, CHUNK), jnp.bfloat16),
            pltpu.VMEM((N_DEV, CHUNK, D_MODEL), jnp.bfloat16),
            pltpu.VMEM((SQ, CHUNK), jnp.bfloat16),
            pltpu.VMEM((SQ, SKV), jnp.float32),
            pltpu.VMEM((2, SKV, CHUNK), jnp.float32),
            pltpu.VMEM((2, SKV, CHUNK), jnp.float32),
            pltpu.SemaphoreType.DMA((N_DEV - 1,)),
            pltpu.SemaphoreType.DMA((N_DEV - 1,)),
            pltpu.SemaphoreType.DMA((N_DEV - 1,)),
            pltpu.SemaphoreType.DMA((N_DEV - 1,)),
            pltpu.SemaphoreType.DMA((2, 2)),
        ],
        compiler_params=pltpu.CompilerParams(
            collective_id=0, vmem_limit_bytes=100 * 1024 * 1024),
    )(x[0], Wq, kf, vf, Wo)
    return out[None]


# device time: 175386 ns/iter; 1.2446x vs baseline; 1.2446x over previous
import jax
import jax.numpy as jnp
from jax import lax
from jax.experimental import pallas as pl
from jax.experimental.pallas import tpu as pltpu

N_DEV = 4
SQ = 1024
SKV = 1024
H_TOT = 32
H_PER = 8
DH = 128
D_MODEL = 1024
SCALE = 0.08838834764831843
QBLK = 64


def _body(x_ref, wq_ref, k_ref, v_ref, wo_ref, out_ref,
          wq_ch, wo_ch, q_buf, ctx_buf, bias_ref, k_stage, v_stage,
          sq_send, sq_recv, so_send, so_recv, kv_sems):
    my = lax.axis_index("i")
    right = lax.rem(my + 1, N_DEV)
    left = lax.rem(my + N_DEV - 1, N_DEV)

    def kv_copies(hg, slot):
        ck = pltpu.make_async_copy(
            k_ref.at[0, :, hg, :], k_stage.at[slot], kv_sems.at[slot, 0])
        cv = pltpu.make_async_copy(
            v_ref.at[0, :, hg, :], v_stage.at[slot], kv_sems.at[slot, 1])
        return ck, cv

    def kv_fetch(hg, slot):
        for c in kv_copies(hg, slot):
            c.start()

    def kv_wait(hg, slot):
        for c in kv_copies(hg, slot):
            c.wait()

    kv_fetch(my * H_PER, 0)

    barrier = pltpu.get_barrier_semaphore()
    for nbr in (left, right):
        pl.semaphore_signal(barrier, inc=1, device_id=(nbr,),
                            device_id_type=pl.DeviceIdType.MESH)
    pl.semaphore_wait(barrier, 2)

    wq_ch[my] = wq_ref[...].astype(jnp.bfloat16)
    wo_ch[my] = wo_ref[...].astype(jnp.bfloat16)

    q_idx = my * SQ + lax.broadcasted_iota(jnp.int32, (SQ, SKV), 0)
    k_idx = lax.broadcasted_iota(jnp.int32, (SQ, SKV), 1)
    qb = q_idx // QBLK
    kb = k_idx // QBLK
    mask = (qb == kb) | (kb == 0) | (lax.rem(qb + kb, 3) == 0)
    bias_ref[...] = jnp.where(mask, 0.0, -1e9).astype(jnp.float32)

    out_ref[...] = jnp.zeros((SQ, D_MODEL), jnp.float32)
    xb = x_ref[...].astype(jnp.bfloat16)

    def compute_chunk(c, c_next, last_chunk=False):
        q_all = lax.dot_general(xb, wq_ch[c], (((1,), (0,)), ((), ())),
                                preferred_element_type=jnp.float32)
        q_buf[...] = (q_all * SCALE).astype(jnp.bfloat16)

        def head_body(hl, _):
            slot = lax.rem(hl, 2)
            hg = c * H_PER + hl
            kv_wait(hg, slot)
            nxt = jnp.where(hl < H_PER - 1, hg + 1, c_next * H_PER)
            if last_chunk:
                @pl.when(hl < H_PER - 1)
                def _():
                    kv_fetch(nxt, lax.rem(hl + 1, 2))
            else:
                kv_fetch(nxt, lax.rem(hl + 1, 2))

            q = q_buf[:, pl.ds(hl * DH, DH)]
            k = k_stage[slot].astype(jnp.bfloat16)
            s = lax.dot_general(q, k, (((1,), (1,)), ((), ())),
                                preferred_element_type=jnp.float32)
            w = jnp.exp(s + bias_ref[...])
            r = 1.0 / jnp.sum(w, axis=-1, keepdims=True)
            v = v_stage[slot].astype(jnp.bfloat16)
            ctx = lax.dot_general(w.astype(jnp.bfloat16), v,
                                  (((1,), (0,)), ((), ())),
                                  preferred_element_type=jnp.float32)
            ctx_buf[:, pl.ds(hl * DH, DH)] = (ctx * r).astype(jnp.bfloat16)
            return 0

        lax.fori_loop(0, H_PER, head_body, 0)
        out_ref[...] += lax.dot_general(ctx_buf[...], wo_ch[c],
                                        (((1,), (0,)), ((), ())),
                                        preferred_element_type=jnp.float32)

    for h in range(N_DEV - 1):
        src = lax.rem(my - h + N_DEV, N_DEV)
        nxt_chunk = lax.rem(my - h - 1 + N_DEV, N_DEV)
        rq = pltpu.make_async_remote_copy(
            src_ref=wq_ch.at[src], dst_ref=wq_ch.at[src],
            send_sem=sq_send.at[h], recv_sem=sq_recv.at[h],
            device_id=(right,), device_id_type=pl.DeviceIdType.MESH)
        ro = pltpu.make_async_remote_copy(
            src_ref=wo_ch.at[src], dst_ref=wo_ch.at[src],
            send_sem=so_send.at[h], recv_sem=so_recv.at[h],
            device_id=(right,), device_id_type=pl.DeviceIdType.MESH)
        rq.start()
        ro.start()
        compute_chunk(src, nxt_chunk)
        rq.wait()
        ro.wait()

    last = lax.rem(my + 1, N_DEV)
    compute_chunk(last, last, last_chunk=True)


def kernel(x, Wq, K_ext, V_ext, Wo):
    vmem = pl.BlockSpec(memory_space=pltpu.MemorySpace.VMEM)
    hbm = pl.BlockSpec(memory_space=pltpu.MemorySpace.HBM)
    out = pl.pallas_call(
        _body,
        out_shape=jax.ShapeDtypeStruct((SQ, D_MODEL), jnp.float32),
        in_specs=[vmem, vmem, hbm, hbm, vmem],
        out_specs=vmem,
        scratch_shapes=[
            pltpu.VMEM((N_DEV, D_MODEL, H_PER * DH), jnp.bfloat16),
            pltpu.VMEM((N_DEV, H_PER * DH, D_MODEL), jnp.bfloat16),
            pltpu.VMEM((SQ, H_PER * DH), jnp.bfloat16),
            pltpu.VMEM((SQ, H_PER * DH), jnp.bfloat16),
            pltpu.VMEM((SQ, SKV), jnp.float32),
            pltpu.VMEM((2, SKV, DH), jnp.float32),
            pltpu.VMEM((2, SKV, DH), jnp.float32),
            pltpu.SemaphoreType.DMA((N_DEV - 1,)),
            pltpu.SemaphoreType.DMA((N_DEV - 1,)),
            pltpu.SemaphoreType.DMA((N_DEV - 1,)),
            pltpu.SemaphoreType.DMA((N_DEV - 1,)),
            pltpu.SemaphoreType.DMA((2, 2)),
        ],
        compiler_params=pltpu.CompilerParams(
            collective_id=0, vmem_limit_bytes=100 * 1024 * 1024),
    )(x[0], Wq, K_ext, V_ext, Wo)
    return out[None]


# device time: 175021 ns/iter; 1.2472x vs baseline; 1.0021x over previous
import jax
import jax.numpy as jnp
from jax import lax
from jax.experimental import pallas as pl
from jax.experimental.pallas import tpu as pltpu

N_DEV = 4
SQ = 1024
SKV = 1024
H_TOT = 32
H_PER = 8
DH = 128
D_MODEL = 1024
SCALE = 0.08838834764831843
QBLK = 64


def _body(x_ref, wq_ref, k_ref, v_ref, wo_ref, out_ref,
          wq_ch, wo_ch, q_buf, ctx_buf, bias_ref, k_stage, v_stage,
          sq_send, sq_recv, so_send, so_recv, kv_sems):
    my = lax.axis_index("i")
    right = lax.rem(my + 1, N_DEV)
    left = lax.rem(my + N_DEV - 1, N_DEV)

    def kv_copies(hg, slot):
        ck = pltpu.make_async_copy(
            k_ref.at[0, :, hg, :], k_stage.at[slot], kv_sems.at[slot, 0])
        cv = pltpu.make_async_copy(
            v_ref.at[0, :, hg, :], v_stage.at[slot], kv_sems.at[slot, 1])
        return ck, cv

    def kv_fetch(hg, slot):
        for c in kv_copies(hg, slot):
            c.start()

    def kv_wait(hg, slot):
        for c in kv_copies(hg, slot):
            c.wait()

    kv_fetch(my * H_PER, 0)

    barrier = pltpu.get_barrier_semaphore()
    for nbr in (left, right):
        pl.semaphore_signal(barrier, inc=1, device_id=(nbr,),
                            device_id_type=pl.DeviceIdType.MESH)
    pl.semaphore_wait(barrier, 2)

    wq_ch[my] = wq_ref[...].astype(jnp.bfloat16)
    wo_ch[my] = wo_ref[...].astype(jnp.bfloat16)

    q_idx = my * SQ + lax.broadcasted_iota(jnp.int32, (SQ, SKV), 0)
    k_idx = lax.broadcasted_iota(jnp.int32, (SQ, SKV), 1)
    qb = q_idx // QBLK
    kb = k_idx // QBLK
    mask = (qb == kb) | (kb == 0) | (lax.rem(qb + kb, 3) == 0)
    bias_ref[...] = jnp.where(mask, 0.0, -1e9).astype(jnp.float32)

    out_ref[...] = jnp.zeros((SQ, D_MODEL), jnp.float32)
    xb = x_ref[...].astype(jnp.bfloat16)

    def compute_chunk(c, c_next, last_chunk=False):
        q_all = lax.dot_general(xb, wq_ch[c], (((1,), (0,)), ((), ())),
                                preferred_element_type=jnp.float32)
        q_buf[...] = (q_all * SCALE).astype(jnp.bfloat16)

        def head_body(hl, _):
            slot = lax.rem(hl, 2)
            hg = c * H_PER + hl
            kv_wait(hg, slot)
            nxt = jnp.where(hl < H_PER - 1, hg + 1, c_next * H_PER)
            if last_chunk:
                @pl.when(hl < H_PER - 1)
                def _():
                    kv_fetch(nxt, lax.rem(hl + 1, 2))
            else:
                kv_fetch(nxt, lax.rem(hl + 1, 2))

            q = q_buf[:, pl.ds(hl * DH, DH)]
            k = k_stage[slot].astype(jnp.bfloat16)
            s = lax.dot_general(q, k, (((1,), (1,)), ((), ())),
                                preferred_element_type=jnp.float32)
            w = jnp.exp(s + bias_ref[...])
            r = 1.0 / jnp.sum(w, axis=-1, keepdims=True)
            v = v_stage[slot].astype(jnp.bfloat16)
            ctx = lax.dot_general(w.astype(jnp.bfloat16), v,
                                  (((1,), (0,)), ((), ())),
                                  preferred_element_type=jnp.float32)
            ctx_buf[:, pl.ds(hl * DH, DH)] = (ctx * r).astype(jnp.bfloat16)
            return 0

        lax.fori_loop(0, H_PER, head_body, 0, unroll=2)
        out_ref[...] += lax.dot_general(ctx_buf[...], wo_ch[c],
                                        (((1,), (0,)), ((), ())),
                                        preferred_element_type=jnp.float32)

    for h in range(N_DEV - 1):
        src = lax.rem(my - h + N_DEV, N_DEV)
        nxt_chunk = lax.rem(my - h - 1 + N_DEV, N_DEV)
        rq = pltpu.make_async_remote_copy(
            src_ref=wq_ch.at[src], dst_ref=wq_ch.at[src],
            send_sem=sq_send.at[h], recv_sem=sq_recv.at[h],
            device_id=(right,), device_id_type=pl.DeviceIdType.MESH)
        ro = pltpu.make_async_remote_copy(
            src_ref=wo_ch.at[src], dst_ref=wo_ch.at[src],
            send_sem=so_send.at[h], recv_sem=so_recv.at[h],
            device_id=(right,), device_id_type=pl.DeviceIdType.MESH)
        rq.start()
        ro.start()
        compute_chunk(src, nxt_chunk)
        rq.wait()
        ro.wait()

    last = lax.rem(my + 1, N_DEV)
    compute_chunk(last, last, last_chunk=True)


def kernel(x, Wq, K_ext, V_ext, Wo):
    vmem = pl.BlockSpec(memory_space=pltpu.MemorySpace.VMEM)
    hbm = pl.BlockSpec(memory_space=pltpu.MemorySpace.HBM)
    out = pl.pallas_call(
        _body,
        out_shape=jax.ShapeDtypeStruct((SQ, D_MODEL), jnp.float32),
        in_specs=[vmem, vmem, hbm, hbm, vmem],
        out_specs=vmem,
        scratch_shapes=[
            pltpu.VMEM((N_DEV, D_MODEL, H_PER * DH), jnp.bfloat16),
            pltpu.VMEM((N_DEV, H_PER * DH, D_MODEL), jnp.bfloat16),
            pltpu.VMEM((SQ, H_PER * DH), jnp.bfloat16),
            pltpu.VMEM((SQ, H_PER * DH), jnp.bfloat16),
            pltpu.VMEM((SQ, SKV), jnp.float32),
            pltpu.VMEM((2, SKV, DH), jnp.float32),
            pltpu.VMEM((2, SKV, DH), jnp.float32),
            pltpu.SemaphoreType.DMA((N_DEV - 1,)),
            pltpu.SemaphoreType.DMA((N_DEV - 1,)),
            pltpu.SemaphoreType.DMA((N_DEV - 1,)),
            pltpu.SemaphoreType.DMA((N_DEV - 1,)),
            pltpu.SemaphoreType.DMA((2, 2)),
        ],
        compiler_params=pltpu.CompilerParams(
            collective_id=0, vmem_limit_bytes=100 * 1024 * 1024),
    )(x[0], Wq, K_ext, V_ext, Wo)
    return out[None]


# device time: 157232 ns/iter; 1.3883x vs baseline; 1.1131x over previous
import jax
import jax.numpy as jnp
from jax import lax
from jax.experimental import pallas as pl
from jax.experimental.pallas import tpu as pltpu

N_DEV = 4
SQ = 1024
SKV = 1024
H_TOT = 32
H_PER = 8
DH = 128
D_MODEL = 1024
SCALE = 0.08838834764831843
QBLK = 64


def _body(x_ref, wq_ref, k_ref, v_ref, wo_ref, out_ref,
          wq_ch, wo_ch, q_buf, ctx_buf, bias_ref, k_stage, v_stage,
          sq_send, sq_recv, so_send, so_recv, kv_sems):
    my = lax.axis_index("i")
    right = lax.rem(my + 1, N_DEV)
    left = lax.rem(my + N_DEV - 1, N_DEV)

    def kv_copies(hg, slot):
        ck = pltpu.make_async_copy(
            k_ref.at[0, :, hg, :], k_stage.at[slot], kv_sems.at[slot, 0])
        cv = pltpu.make_async_copy(
            v_ref.at[0, :, hg, :], v_stage.at[slot], kv_sems.at[slot, 1])
        return ck, cv

    def kv_fetch(hg, slot):
        for c in kv_copies(hg, slot):
            c.start()

    def kv_wait(hg, slot):
        for c in kv_copies(hg, slot):
            c.wait()

    kv_fetch(my * H_PER, 0)

    barrier = pltpu.get_barrier_semaphore()
    for nbr in (left, right):
        pl.semaphore_signal(barrier, inc=1, device_id=(nbr,),
                            device_id_type=pl.DeviceIdType.MESH)
    pl.semaphore_wait(barrier, 2)

    wq_ch[my] = wq_ref[...].astype(jnp.bfloat16)
    wo_ch[my] = wo_ref[...].astype(jnp.bfloat16)

    q_idx = my * SQ + lax.broadcasted_iota(jnp.int32, (SQ, SKV), 0)
    k_idx = lax.broadcasted_iota(jnp.int32, (SQ, SKV), 1)
    qb = q_idx // QBLK
    kb = k_idx // QBLK
    mask = (qb == kb) | (kb == 0) | (lax.rem(qb + kb, 3) == 0)
    bias_ref[...] = jnp.where(mask, 0.0, -1e9).astype(jnp.float32)

    xb = x_ref[...].astype(jnp.bfloat16)

    def compute_chunk(c, c_next, last_chunk=False, first_chunk=False,
                      wo_ready=None):
        q_all = lax.dot_general(xb, wq_ch[c], (((1,), (0,)), ((), ())),
                                preferred_element_type=jnp.float32)
        q_buf[...] = (q_all * SCALE).astype(jnp.bfloat16)

        def head_body(hl, _):
            slot = lax.rem(hl, 2)
            hg = c * H_PER + hl
            kv_wait(hg, slot)
            nxt = jnp.where(hl < H_PER - 1, hg + 1, c_next * H_PER)
            if last_chunk:
                @pl.when(hl < H_PER - 1)
                def _():
                    kv_fetch(nxt, lax.rem(hl + 1, 2))
            else:
                kv_fetch(nxt, lax.rem(hl + 1, 2))

            q = q_buf[:, pl.ds(hl * DH, DH)]
            k = k_stage[slot].astype(jnp.bfloat16)
            s = lax.dot_general(q, k, (((1,), (1,)), ((), ())),
                                preferred_element_type=jnp.float32)
            w = jnp.exp(s + bias_ref[...])
            r = 1.0 / jnp.sum(w, axis=-1, keepdims=True)
            v = v_stage[slot].astype(jnp.bfloat16)
            ctx = lax.dot_general(w.astype(jnp.bfloat16), v,
                                  (((1,), (0,)), ((), ())),
                                  preferred_element_type=jnp.float32)
            ctx_buf[:, pl.ds(hl * DH, DH)] = (ctx * r).astype(jnp.bfloat16)
            return 0

        lax.fori_loop(0, H_PER, head_body, 0, unroll=2)
        if wo_ready is not None:
            wo_ready.wait()
        proj = lax.dot_general(ctx_buf[...], wo_ch[c],
                               (((1,), (0,)), ((), ())),
                               preferred_element_type=jnp.float32)
        if first_chunk:
            out_ref[...] = proj
        else:
            out_ref[...] += proj

    wo_pending = None
    for h in range(N_DEV - 1):
        src = lax.rem(my - h + N_DEV, N_DEV)
        nxt_chunk = lax.rem(my - h - 1 + N_DEV, N_DEV)
        rq = pltpu.make_async_remote_copy(
            src_ref=wq_ch.at[src], dst_ref=wq_ch.at[src],
            send_sem=sq_send.at[h], recv_sem=sq_recv.at[h],
            device_id=(right,), device_id_type=pl.DeviceIdType.MESH)
        ro = pltpu.make_async_remote_copy(
            src_ref=wo_ch.at[src], dst_ref=wo_ch.at[src],
            send_sem=so_send.at[h], recv_sem=so_recv.at[h],
            device_id=(right,), device_id_type=pl.DeviceIdType.MESH)
        rq.start()
        ro.start()
        compute_chunk(src, nxt_chunk, first_chunk=(h == 0),
                      wo_ready=wo_pending)
        rq.wait()
        wo_pending = ro

    last = lax.rem(my + 1, N_DEV)
    compute_chunk(last, last, last_chunk=True, wo_ready=wo_pending)


def kernel(x, Wq, K_ext, V_ext, Wo):
    vmem = pl.BlockSpec(memory_space=pltpu.MemorySpace.VMEM)
    hbm = pl.BlockSpec(memory_space=pltpu.MemorySpace.HBM)
    out = pl.pallas_call(
        _body,
        out_shape=jax.ShapeDtypeStruct((SQ, D_MODEL), jnp.float32),
        in_specs=[vmem, vmem, hbm, hbm, vmem],
        out_specs=vmem,
        scratch_shapes=[
            pltpu.VMEM((N_DEV, D_MODEL, H_PER * DH), jnp.bfloat16),
            pltpu.VMEM((N_DEV, H_PER * DH, D_MODEL), jnp.bfloat16),
            pltpu.VMEM((SQ, H_PER * DH), jnp.bfloat16),
            pltpu.VMEM((SQ, H_PER * DH), jnp.bfloat16),
            pltpu.VMEM((SQ, SKV), jnp.float32),
            pltpu.VMEM((2, SKV, DH), jnp.float32),
            pltpu.VMEM((2, SKV, DH), jnp.float32),
            pltpu.SemaphoreType.DMA((N_DEV - 1,)),
            pltpu.SemaphoreType.DMA((N_DEV - 1,)),
            pltpu.SemaphoreType.DMA((N_DEV - 1,)),
            pltpu.SemaphoreType.DMA((N_DEV - 1,)),
            pltpu.SemaphoreType.DMA((2, 2)),
        ],
        compiler_params=pltpu.CompilerParams(
            collective_id=0, vmem_limit_bytes=100 * 1024 * 1024),
    )(x[0], Wq, K_ext, V_ext, Wo)
    return out[None]


# device time: 154003 ns/iter; 1.4174x vs baseline; 1.0210x over previous
import jax
import jax.numpy as jnp
from jax import lax
from jax.experimental import pallas as pl
from jax.experimental.pallas import tpu as pltpu

N_DEV = 4
SQ = 1024
SKV = 1024
H_TOT = 32
H_PER = 8
DH = 128
D_MODEL = 1024
SCALE = 0.08838834764831843
QBLK = 64


def _body(x_ref, wq_ref, k_ref, v_ref, wo_ref, out_ref,
          wq_ch, wo_ch, q_buf, ctx_buf, bias_ref, k_stage, v_stage,
          sq_send, sq_recv, so_send, so_recv, kv_sems):
    my = lax.axis_index("i")
    right = lax.rem(my + 1, N_DEV)
    left = lax.rem(my + N_DEV - 1, N_DEV)

    def kv_copies(hg, slot):
        ck = pltpu.make_async_copy(
            k_ref.at[0, :, hg, :], k_stage.at[slot], kv_sems.at[slot, 0])
        cv = pltpu.make_async_copy(
            v_ref.at[0, :, hg, :], v_stage.at[slot], kv_sems.at[slot, 1])
        return ck, cv

    def kv_fetch(hg, slot):
        for c in kv_copies(hg, slot):
            c.start()

    def kv_wait(hg, slot):
        for c in kv_copies(hg, slot):
            c.wait()

    kv_fetch(my * H_PER, 0)

    barrier = pltpu.get_barrier_semaphore()
    for nbr in (left, right):
        pl.semaphore_signal(barrier, inc=1, device_id=(nbr,),
                            device_id_type=pl.DeviceIdType.MESH)
    pl.semaphore_wait(barrier, 2)

    wq_ch[my] = wq_ref[...].astype(jnp.bfloat16)
    wo_ch[my] = wo_ref[...].astype(jnp.bfloat16)

    def hop_rdmas(h, src):
        rq = pltpu.make_async_remote_copy(
            src_ref=wq_ch.at[src], dst_ref=wq_ch.at[src],
            send_sem=sq_send.at[h], recv_sem=sq_recv.at[h],
            device_id=(right,), device_id_type=pl.DeviceIdType.MESH)
        ro = pltpu.make_async_remote_copy(
            src_ref=wo_ch.at[src], dst_ref=wo_ch.at[src],
            send_sem=so_send.at[h], recv_sem=so_recv.at[h],
            device_id=(right,), device_id_type=pl.DeviceIdType.MESH)
        rq.start()
        ro.start()
        return rq, ro

    hop0 = hop_rdmas(0, my)

    q_idx = my * SQ + lax.broadcasted_iota(jnp.int32, (SQ, SKV), 0)
    k_idx = lax.broadcasted_iota(jnp.int32, (SQ, SKV), 1)
    qb = q_idx // QBLK
    kb = k_idx // QBLK
    mask = (qb == kb) | (kb == 0) | (lax.rem(qb + kb, 3) == 0)
    bias_ref[...] = jnp.where(mask, 0.0, -1e9).astype(jnp.float32)

    xb = x_ref[...].astype(jnp.bfloat16)

    def compute_chunk(c, c_next, last_chunk=False, first_chunk=False,
                      wo_ready=None):
        q_all = lax.dot_general(xb, wq_ch[c], (((1,), (0,)), ((), ())),
                                preferred_element_type=jnp.float32)
        q_buf[...] = (q_all * SCALE).astype(jnp.bfloat16)

        def head_body(hl, _):
            slot = lax.rem(hl, 2)
            hg = c * H_PER + hl
            kv_wait(hg, slot)
            nxt = jnp.where(hl < H_PER - 1, hg + 1, c_next * H_PER)
            if last_chunk:
                @pl.when(hl < H_PER - 1)
                def _():
                    kv_fetch(nxt, lax.rem(hl + 1, 2))
            else:
                kv_fetch(nxt, lax.rem(hl + 1, 2))

            q = q_buf[:, pl.ds(hl * DH, DH)]
            k = k_stage[slot].astype(jnp.bfloat16)
            s = lax.dot_general(q, k, (((1,), (1,)), ((), ())),
                                preferred_element_type=jnp.float32)
            w = jnp.exp(s + bias_ref[...])
            r = 1.0 / jnp.sum(w, axis=-1, keepdims=True)
            v = v_stage[slot].astype(jnp.bfloat16)
            ctx = lax.dot_general(w.astype(jnp.bfloat16), v,
                                  (((1,), (0,)), ((), ())),
                                  preferred_element_type=jnp.float32)
            ctx_buf[:, pl.ds(hl * DH, DH)] = (ctx * r).astype(jnp.bfloat16)
            return 0

        lax.fori_loop(0, H_PER, head_body, 0, unroll=2)
        if wo_ready is not None:
            wo_ready.wait()
        proj = lax.dot_general(ctx_buf[...], wo_ch[c],
                               (((1,), (0,)), ((), ())),
                               preferred_element_type=jnp.float32)
        if first_chunk:
            out_ref[...] = proj
        else:
            out_ref[...] += proj

    wo_pending = None
    for h in range(N_DEV - 1):
        src = lax.rem(my - h + N_DEV, N_DEV)
        nxt_chunk = lax.rem(my - h - 1 + N_DEV, N_DEV)
        rq, ro = hop0 if h == 0 else hop_rdmas(h, src)
        compute_chunk(src, nxt_chunk, first_chunk=(h == 0),
                      wo_ready=wo_pending)
        rq.wait()
        wo_pending = ro

    last = lax.rem(my + 1, N_DEV)
    compute_chunk(last, last, last_chunk=True, wo_ready=wo_pending)


def kernel(x, Wq, K_ext, V_ext, Wo):
    vmem = pl.BlockSpec(memory_space=pltpu.MemorySpace.VMEM)
    hbm = pl.BlockSpec(memory_space=pltpu.MemorySpace.HBM)
    out = pl.pallas_call(
        _body,
        out_shape=jax.ShapeDtypeStruct((SQ, D_MODEL), jnp.float32),
        in_specs=[vmem, vmem, hbm, hbm, vmem],
        out_specs=vmem,
        scratch_shapes=[
            pltpu.VMEM((N_DEV, D_MODEL, H_PER * DH), jnp.bfloat16),
            pltpu.VMEM((N_DEV, H_PER * DH, D_MODEL), jnp.bfloat16),
            pltpu.VMEM((SQ, H_PER * DH), jnp.bfloat16),
            pltpu.VMEM((SQ, H_PER * DH), jnp.bfloat16),
            pltpu.VMEM((SQ, SKV), jnp.float32),
            pltpu.VMEM((2, SKV, DH), jnp.float32),
            pltpu.VMEM((2, SKV, DH), jnp.float32),
            pltpu.SemaphoreType.DMA((N_DEV - 1,)),
            pltpu.SemaphoreType.DMA((N_DEV - 1,)),
            pltpu.SemaphoreType.DMA((N_DEV - 1,)),
            pltpu.SemaphoreType.DMA((N_DEV - 1,)),
            pltpu.SemaphoreType.DMA((2, 2)),
        ],
        compiler_params=pltpu.CompilerParams(
            collective_id=0, vmem_limit_bytes=100 * 1024 * 1024),
    )(x[0], Wq, K_ext, V_ext, Wo)
    return out[None]
